# baseline (speedup 1.0000x reference)
import jax
from jax.experimental import pallas as pl
from jax.experimental.pallas import tpu as pltpu

_N_CHUNKS = 8
_ROWS = 10000
_CHUNK = _ROWS // _N_CHUNKS


def _copy_kernel(x_ref, o_ref, buf0, buf1, si0, si1, so0, so1):
    bufs = (buf0, buf1)
    sin = (si0, si1)
    sout = (so0, so1)
    out_copies = [None, None]
    for i in range(_N_CHUNKS):
        b = i % 2
        if out_copies[b] is not None:
            out_copies[b].wait()
        sl = pl.ds(i * _CHUNK, _CHUNK)
        cin = pltpu.make_async_copy(x_ref.at[sl], bufs[b], sin[b])
        cin.start()
        cin.wait()
        cout = pltpu.make_async_copy(bufs[b], o_ref.at[sl], sout[b])
        cout.start()
        out_copies[b] = cout
    for c in out_copies:
        c.wait()


def kernel(x, edge_index):
    del edge_index
    n, d = x.shape
    return pl.pallas_call(
        _copy_kernel,
        out_shape=jax.ShapeDtypeStruct(x.shape, x.dtype),
        in_specs=[pl.BlockSpec(memory_space=pl.ANY)],
        out_specs=pl.BlockSpec(memory_space=pl.ANY),
        scratch_shapes=[
            pltpu.MemorySpace.VMEM((_CHUNK, d), x.dtype),
            pltpu.MemorySpace.VMEM((_CHUNK, d), x.dtype),
            pltpu.SemaphoreType.DMA,
            pltpu.SemaphoreType.DMA,
            pltpu.SemaphoreType.DMA,
            pltpu.SemaphoreType.DMA,
        ],
    )(x)


# speedup vs baseline: 2.8304x; 2.8304x over previous
import jax
from jax.experimental import pallas as pl
from jax.experimental.pallas import tpu as pltpu

_BLOCK_ROWS = 5000


def _copy_kernel(x_ref, o_ref):
    o_ref[...] = x_ref[...]


def kernel(x, edge_index):
    del edge_index
    n, d = x.shape
    grid = (pl.cdiv(n, _BLOCK_ROWS),)
    return pl.pallas_call(
        _copy_kernel,
        out_shape=jax.ShapeDtypeStruct(x.shape, x.dtype),
        grid=grid,
        in_specs=[pl.BlockSpec((_BLOCK_ROWS, d), lambda i: (i, 0))],
        out_specs=pl.BlockSpec((_BLOCK_ROWS, d), lambda i: (i, 0)),
        compiler_params=pltpu.CompilerParams(
            dimension_semantics=("arbitrary",),
        ),
    )(x)
